# Initial kernel scaffold; baseline (speedup 1.0000x reference)
import jax
import jax.numpy as jnp
from jax.experimental import pallas as pl

D = 8192
K = 32
ROWS_PER_BLOCK = 128


def _kernel(w_ref, soft_ref, hard_ref):
    pid = pl.program_id(0)
    w = w_ref[...]
    r, d = w.shape

    row_ids = pid * r + jax.lax.broadcasted_iota(jnp.int32, (r, d), 0)
    col_ids = jax.lax.broadcasted_iota(jnp.int32, (r, d), 1)
    w = jnp.where(col_ids == row_ids, jnp.float32(0.0), w)

    soft_ref[...] = 1.0 / (1.0 + jnp.exp(w * -5.0))

    b = jax.lax.bitcast_convert_type(w, jnp.int32)
    key = b ^ ((b >> 31) & jnp.int32(0x7FFFFFFF))

    lo = jnp.full((r, 1), jnp.int32(-2147483648))
    for j in range(30, -1, -1):
        cand = lo | jnp.int32(1 << j)
        cnt = jnp.sum((key >= cand).astype(jnp.int32), axis=1, keepdims=True)
        lo = jnp.where(cnt >= K, cand, lo)

    hard = (key >= lo) & (w > 0.0)
    hard_ref[...] = hard.astype(jnp.float32)


@jax.jit
def kernel(W_raw):
    grid = (D // ROWS_PER_BLOCK,)
    soft, hard = pl.pallas_call(
        _kernel,
        grid=grid,
        in_specs=[pl.BlockSpec((ROWS_PER_BLOCK, D), lambda i: (i, 0))],
        out_specs=[
            pl.BlockSpec((ROWS_PER_BLOCK, D), lambda i: (i, 0)),
            pl.BlockSpec((ROWS_PER_BLOCK, D), lambda i: (i, 0)),
        ],
        out_shape=[
            jax.ShapeDtypeStruct((D, D), jnp.float32),
            jax.ShapeDtypeStruct((D, D), jnp.float32),
        ],
    )(W_raw)
    return (soft, hard)


# speedup vs baseline: 7.0090x; 7.0090x over previous
import jax
import jax.numpy as jnp
from jax.experimental import pallas as pl

D = 8192
K = 32
ROWS_PER_BLOCK = 128


def _kernel(w_ref, soft_ref, hard_ref):
    pid = pl.program_id(0)
    w = w_ref[...]
    r, d = w.shape

    row_ids = pid * r + jax.lax.broadcasted_iota(jnp.int32, (r, d), 0)
    col_ids = jax.lax.broadcasted_iota(jnp.int32, (r, d), 1)
    w = jnp.where(col_ids == row_ids, jnp.float32(0.0), w)

    soft_ref[...] = 1.0 / (1.0 + jnp.exp(w * -5.0))

    b = jax.lax.bitcast_convert_type(w, jnp.int32)
    key = b ^ ((b >> 31) & jnp.int32(0x7FFFFFFF))

    lo = jnp.full((r, 1), jnp.int32(-2147483648))
    cnt0 = jnp.sum((key >= 0).astype(jnp.int32), axis=1, keepdims=True)
    lo = jnp.where(cnt0 >= K, jnp.int32(0), lo)
    for j in range(30, -1, -1):
        cand = lo | jnp.int32(1 << j)
        cnt = jnp.sum((key >= cand).astype(jnp.int32), axis=1, keepdims=True)
        lo = jnp.where(cnt >= K, cand, lo)

    hard = (key >= lo) & (w > 0.0)
    hard_ref[...] = hard.astype(jnp.float32)


@jax.jit
def kernel(W_raw):
    grid = (D // ROWS_PER_BLOCK,)
    soft, hard = pl.pallas_call(
        _kernel,
        grid=grid,
        in_specs=[pl.BlockSpec((ROWS_PER_BLOCK, D), lambda i: (i, 0))],
        out_specs=[
            pl.BlockSpec((ROWS_PER_BLOCK, D), lambda i: (i, 0)),
            pl.BlockSpec((ROWS_PER_BLOCK, D), lambda i: (i, 0)),
        ],
        out_shape=[
            jax.ShapeDtypeStruct((D, D), jnp.float32),
            jax.ShapeDtypeStruct((D, D), jnp.float32),
        ],
    )(W_raw)
    return (soft, hard)
